# baseline (speedup 1.0000x reference)
import functools

import jax
import jax.numpy as jnp
from jax import lax
from jax.experimental import pallas as pl
from jax.experimental.pallas import tpu as pltpu
from jax.experimental.pallas import tpu_sc as plsc

BATCH = 16384
HIDDEN = 64
NROWS = 1000000
G = 8192
NGM = NROWS // (2 * G)
SPLIT = NGM * 2 * G
NTAIL = NROWS - SPLIT
TAILBASE = NGM * G
PROWS = (NGM + 1) * G


def _repack_body(x1_ref, x2_ref, tail_ref, out_ref):
    j = pl.program_id(0)

    @pl.when(j < NGM)
    def _():
        eye = jnp.eye(HIDDEN, dtype=jnp.float32)
        dn = (((0,), (0,)), ((), ()))
        t1 = lax.dot_general(x1_ref[...], eye, dn,
                             preferred_element_type=jnp.float32)
        t2 = lax.dot_general(x2_ref[...], eye, dn,
                             preferred_element_type=jnp.float32)
        out_ref[...] = jnp.concatenate([t1, t2], axis=1).astype(jnp.bfloat16)

    @pl.when(j == NGM)
    def _():
        out_ref[...] = tail_ref[...].astype(jnp.bfloat16)


def _repack(t_t, tail_pk):
    clamp = lambda b: jnp.minimum(b, 2 * NGM - 1)
    return pl.pallas_call(
        _repack_body,
        grid=(NGM + 1,),
        in_specs=[
            pl.BlockSpec((HIDDEN, G), lambda j: (0, clamp(2 * j))),
            pl.BlockSpec((HIDDEN, G), lambda j: (0, clamp(2 * j + 1))),
            pl.BlockSpec((G, 2 * HIDDEN), lambda j: (0, 0)),
        ],
        out_specs=pl.BlockSpec((G, 2 * HIDDEN), lambda j: (j, 0)),
        out_shape=jax.ShapeDtypeStruct((PROWS, 2 * HIDDEN), jnp.bfloat16),
    )(t_t, t_t, tail_pk)


def _sc_gather(up, ip, user_id, item_id):
    info = plsc.get_sparse_core_info()
    nc, ns = info.num_cores, info.num_subcores
    nw = nc * ns
    bpw = BATCH // nw
    lanes = 16
    nvec = bpw // lanes
    qchunk = 128
    nq = bpw // qchunk
    mesh = plsc.VectorSubcoreMesh(core_axis_name="c", subcore_axis_name="s")

    @functools.partial(
        pl.kernel,
        mesh=mesh,
        compiler_params=pltpu.CompilerParams(use_tc_tiling_on_sc=False),
        out_type=[
            jax.ShapeDtypeStruct((BATCH, 2 * HIDDEN), jnp.bfloat16),
            jax.ShapeDtypeStruct((BATCH, 2 * HIDDEN), jnp.bfloat16),
        ],
        scratch_types=[
            pltpu.VMEM((bpw,), jnp.int32),
            pltpu.VMEM((bpw,), jnp.int32),
            pltpu.VMEM((bpw, 2 * HIDDEN), jnp.bfloat16),
            pltpu.SemaphoreType.DMA,
        ],
    )
    def k(upt, ipt, uid, iid, out_u, out_i, idx_v, p_v, buf, sem):
        wid = lax.axis_index("s") * nc + lax.axis_index("c")
        base = wid * bpw

        def one_table(table, ids_hbm, out_hbm):
            pltpu.sync_copy(ids_hbm.at[pl.ds(base, bpw)], idx_v)

            def fold(m, carry):
                v = idx_v[pl.ds(m * lanes, lanes)]
                p = ((v >> 14) << 13) + (v & (G - 1))
                p_v[pl.ds(m * lanes, lanes)] = jnp.where(
                    v >= SPLIT, v - SPLIT + TAILBASE, p)
                return carry

            lax.fori_loop(0, nvec, fold, 0)
            pltpu.async_copy(table.at[p_v], buf, sem).wait()
            pltpu.sync_copy(buf, out_hbm.at[pl.ds(base, bpw)])

        one_table(upt, uid, out_u)
        one_table(ipt, iid, out_i)

    return k(up, ip, user_id, item_id)


def _mlp_body(u_ref, i_ref, um_ref, im_ref, w1a, w1b, b1r, w2, b2r, w3, b3r,
              out_ref):
    d = HIDDEN
    upk = u_ref[...].astype(jnp.float32)
    ipk = i_ref[...].astype(jnp.float32)
    u = jnp.where(um_ref[...] > 0.5, upk[:, d:], upk[:, :d])
    i = jnp.where(im_ref[...] > 0.5, ipk[:, d:], ipk[:, :d])
    h = jnp.dot(u, w1a[...], preferred_element_type=jnp.float32)
    h = h + jnp.dot(i, w1b[...], preferred_element_type=jnp.float32)
    h = jnp.maximum(h + b1r[...], 0.0)
    h = jnp.dot(h, w2[...], preferred_element_type=jnp.float32) + b2r[...]
    h = jnp.maximum(h, 0.0)
    out_ref[...] = jnp.dot(h, w3[...], preferred_element_type=jnp.float32) + b3r[...]


def _mlp(upk, ipk, umask, imask, W1, b1, W2, b2, W3, b3):
    d = HIDDEN
    rows = 2048
    grid = (BATCH // rows,)
    full = lambda shape: pl.BlockSpec(shape, lambda r: (0, 0))
    return pl.pallas_call(
        _mlp_body,
        grid=grid,
        in_specs=[
            pl.BlockSpec((rows, 2 * d), lambda r: (r, 0)),
            pl.BlockSpec((rows, 2 * d), lambda r: (r, 0)),
            pl.BlockSpec((rows, 1), lambda r: (r, 0)),
            pl.BlockSpec((rows, 1), lambda r: (r, 0)),
            full((d, d)),
            full((d, d)),
            full((1, d)),
            full((d, d // 2)),
            full((1, d // 2)),
            full((d // 2, d // 4)),
            full((1, d // 4)),
        ],
        out_specs=pl.BlockSpec((rows, d // 4), lambda r: (r, 0)),
        out_shape=jax.ShapeDtypeStruct((BATCH, d // 4), jnp.float32),
    )(upk, ipk, umask, imask, W1[:d], W1[d:], b1.reshape(1, d), W2,
      b2.reshape(1, d // 2), W3, b3.reshape(1, d // 4))


def kernel(user_id, item_id, user_table, item_table, W1, b1, W2, b2, W3, b3):
    uid = user_id.astype(jnp.int32)
    iid = item_id.astype(jnp.int32)

    def tail_pack(table):
        tail_rows = table[SPLIT:]
        tail_pk = jnp.concatenate(
            [tail_rows, jnp.zeros((NTAIL, HIDDEN), jnp.float32)], axis=1)
        return jnp.pad(tail_pk, ((0, G - NTAIL), (0, 0)))

    up = _repack(user_table.T, tail_pack(user_table))
    ip = _repack(item_table.T, tail_pack(item_table))
    upk, ipk = _sc_gather(up, ip, uid, iid)

    def half_mask(v):
        m = jnp.where(v >= SPLIT, 0, (v >> 13) & 1)
        return m.astype(jnp.float32).reshape(BATCH, 1)

    return _mlp(upk, ipk, half_mask(uid), half_mask(iid),
                W1, b1, W2, b2, W3, b3)


# speedup vs baseline: 2.4654x; 2.4654x over previous
import functools

import jax
import jax.numpy as jnp
from jax import lax
from jax.experimental import pallas as pl
from jax.experimental.pallas import tpu as pltpu
from jax.experimental.pallas import tpu_sc as plsc

BATCH = 16384
HIDDEN = 64
NROWS = 1000000
G = 8192
NGM = NROWS // (2 * G)
SPLIT = NGM * 2 * G
NTAIL = NROWS - SPLIT
TAILBASE = NGM * G
PROWS = (NGM + 1) * G


def _repack_body(x1_ref, x2_ref, tail_ref, out_ref):
    j = pl.program_id(0)

    @pl.when(j < NGM)
    def _():
        eye = jnp.eye(HIDDEN, dtype=jnp.float32)
        dn = (((0,), (0,)), ((), ()))
        t1 = lax.dot_general(x1_ref[...], eye, dn,
                             preferred_element_type=jnp.float32)
        t2 = lax.dot_general(x2_ref[...], eye, dn,
                             preferred_element_type=jnp.float32)
        out_ref[...] = jnp.concatenate([t1, t2], axis=1)

    @pl.when(j == NGM)
    def _():
        out_ref[...] = tail_ref[...]


def _repack(t_t, tail_pk):
    clamp = lambda b: jnp.minimum(b, 2 * NGM - 1)
    return pl.pallas_call(
        _repack_body,
        grid=(NGM + 1,),
        in_specs=[
            pl.BlockSpec((HIDDEN, G), lambda j: (0, clamp(2 * j))),
            pl.BlockSpec((HIDDEN, G), lambda j: (0, clamp(2 * j + 1))),
            pl.BlockSpec((G, 2 * HIDDEN), lambda j: (0, 0)),
        ],
        out_specs=pl.BlockSpec((G, 2 * HIDDEN), lambda j: (j, 0)),
        out_shape=jax.ShapeDtypeStruct((PROWS, 2 * HIDDEN), jnp.float32),
    )(t_t, t_t, tail_pk)


def _sc_gather(up, ip, user_id, item_id):
    info = plsc.get_sparse_core_info()
    nc, ns = info.num_cores, info.num_subcores
    nw = nc * ns
    bpw = BATCH // nw
    lanes = 16
    nvec = bpw // lanes
    qchunk = 128
    nq = bpw // qchunk
    mesh = plsc.VectorSubcoreMesh(core_axis_name="c", subcore_axis_name="s")

    @functools.partial(
        pl.kernel,
        mesh=mesh,
        compiler_params=pltpu.CompilerParams(use_tc_tiling_on_sc=False),
        out_type=[
            jax.ShapeDtypeStruct((BATCH, 2 * HIDDEN), jnp.float32),
            jax.ShapeDtypeStruct((BATCH, 2 * HIDDEN), jnp.float32),
        ],
        scratch_types=[
            pltpu.VMEM((bpw,), jnp.int32),
            pltpu.VMEM((bpw,), jnp.int32),
            pltpu.VMEM((bpw, 2 * HIDDEN), jnp.float32),
            pltpu.SemaphoreType.DMA,
        ],
    )
    def k(upt, ipt, uid, iid, out_u, out_i, idx_v, p_v, buf, sem):
        wid = lax.axis_index("s") * nc + lax.axis_index("c")
        base = wid * bpw

        def one_table(table, ids_hbm, out_hbm):
            pltpu.sync_copy(ids_hbm.at[pl.ds(base, bpw)], idx_v)

            def fold(m, carry):
                v = idx_v[pl.ds(m * lanes, lanes)]
                p = ((v >> 14) << 13) + (v & (G - 1))
                p_v[pl.ds(m * lanes, lanes)] = jnp.where(
                    v >= SPLIT, v - SPLIT + TAILBASE, p)
                return carry

            lax.fori_loop(0, nvec, fold, 0)
            pltpu.async_copy(table.at[p_v], buf, sem).wait()
            pltpu.sync_copy(buf, out_hbm.at[pl.ds(base, bpw)])

        one_table(upt, uid, out_u)
        one_table(ipt, iid, out_i)

    return k(up, ip, user_id, item_id)


def _mlp_body(u_ref, i_ref, um_ref, im_ref, w1a, w1b, b1r, w2, b2r, w3, b3r,
              out_ref):
    d = HIDDEN
    upk = u_ref[...]
    ipk = i_ref[...]
    u = jnp.where(um_ref[...] > 0.5, upk[:, d:], upk[:, :d])
    i = jnp.where(im_ref[...] > 0.5, ipk[:, d:], ipk[:, :d])
    h = jnp.dot(u, w1a[...], preferred_element_type=jnp.float32)
    h = h + jnp.dot(i, w1b[...], preferred_element_type=jnp.float32)
    h = jnp.maximum(h + b1r[...], 0.0)
    h = jnp.dot(h, w2[...], preferred_element_type=jnp.float32) + b2r[...]
    h = jnp.maximum(h, 0.0)
    out_ref[...] = jnp.dot(h, w3[...], preferred_element_type=jnp.float32) + b3r[...]


def _mlp(upk, ipk, umask, imask, W1, b1, W2, b2, W3, b3):
    d = HIDDEN
    rows = 2048
    grid = (BATCH // rows,)
    full = lambda shape: pl.BlockSpec(shape, lambda r: (0, 0))
    return pl.pallas_call(
        _mlp_body,
        grid=grid,
        in_specs=[
            pl.BlockSpec((rows, 2 * d), lambda r: (r, 0)),
            pl.BlockSpec((rows, 2 * d), lambda r: (r, 0)),
            pl.BlockSpec((rows, 1), lambda r: (r, 0)),
            pl.BlockSpec((rows, 1), lambda r: (r, 0)),
            full((d, d)),
            full((d, d)),
            full((1, d)),
            full((d, d // 2)),
            full((1, d // 2)),
            full((d // 2, d // 4)),
            full((1, d // 4)),
        ],
        out_specs=pl.BlockSpec((rows, d // 4), lambda r: (r, 0)),
        out_shape=jax.ShapeDtypeStruct((BATCH, d // 4), jnp.float32),
    )(upk, ipk, umask, imask, W1[:d], W1[d:], b1.reshape(1, d), W2,
      b2.reshape(1, d // 2), W3, b3.reshape(1, d // 4))


def kernel(user_id, item_id, user_table, item_table, W1, b1, W2, b2, W3, b3):
    uid = user_id.astype(jnp.int32)
    iid = item_id.astype(jnp.int32)

    def tail_pack(table):
        tail_rows = table[SPLIT:]
        tail_pk = jnp.concatenate(
            [tail_rows, jnp.zeros((NTAIL, HIDDEN), jnp.float32)], axis=1)
        return jnp.pad(tail_pk, ((0, G - NTAIL), (0, 0)))

    up = _repack(user_table.T, tail_pack(user_table))
    ip = _repack(item_table.T, tail_pack(item_table))
    upk, ipk = _sc_gather(up, ip, uid, iid)

    def half_mask(v):
        m = jnp.where(v >= SPLIT, 0, (v >> 13) & 1)
        return m.astype(jnp.float32).reshape(BATCH, 1)

    return _mlp(upk, ipk, half_mask(uid), half_mask(iid),
                W1, b1, W2, b2, W3, b3)
